# baseline (speedup 1.0000x reference)
import functools

import jax
import jax.numpy as jnp
from jax import lax
from jax.experimental import pallas as pl
from jax.experimental.pallas import tpu as pltpu
from jax.experimental.pallas import tpu_sc as plsc

D_MODEL = 64
PAD_W = 128
NUM_CORES = 2
NUM_SUBCORES = 16
NW = NUM_CORES * NUM_SUBCORES


@functools.lru_cache(maxsize=None)
def _make_gather(BT: int, S: int, chunk: int):
    B = BT * S
    assert B % (NW * chunk) == 0 and chunk % S == 0
    b_per_w = B // NW
    n_chunks = b_per_w // chunk
    mesh = plsc.VectorSubcoreMesh(
        core_axis_name="c", subcore_axis_name="s",
        num_cores=NUM_CORES, num_subcores=NUM_SUBCORES)

    @functools.partial(
        pl.kernel,
        out_type=jax.ShapeDtypeStruct((BT, S, D_MODEL), jnp.float32),
        mesh=mesh,
        compiler_params=pltpu.CompilerParams(use_tc_tiling_on_sc=False),
        scratch_types=[
            pltpu.VMEM((b_per_w,), jnp.int32),
            pltpu.VMEM((chunk, PAD_W), jnp.float32),
            pltpu.VMEM((chunk, PAD_W), jnp.float32),
            pltpu.SemaphoreType.DMA,
            pltpu.SemaphoreType.DMA,
            pltpu.SemaphoreType.DMA,
            pltpu.SemaphoreType.DMA,
        ],
    )
    def k(idx_hbm, table_hbm, out_hbm,
          idx_v, rows0, rows1, g0, g1, w0, w1):
        wid = lax.axis_index("s") * NUM_CORES + lax.axis_index("c")
        base = wid * b_per_w
        rows = [rows0, rows1]
        gsem = [g0, g1]
        wsem = [w0, w1]

        pltpu.sync_copy(idx_hbm.at[pl.ds(base, b_per_w)], idx_v)

        def start_gather(c, b):
            return pltpu.async_copy(
                table_hbm.at[idx_v.at[pl.ds(c * chunk, chunk)]],
                rows[b], gsem[b])

        gh = [None, None]
        wh = [None, None]
        for c in range(min(2, n_chunks)):
            gh[c] = start_gather(c, c)
        for c in range(n_chunks):
            b = c & 1
            gh[b].wait()
            bbase = (base + c * chunk) // S
            wh[b] = [
                pltpu.async_copy(
                    rows[b].at[pl.ds(j * S, S), pl.ds(0, D_MODEL)],
                    out_hbm.at[bbase + j], wsem[b])
                for j in range(chunk // S)
            ]
            if c + 2 < n_chunks:
                for h in wh[b]:
                    h.wait()
                gh[b] = start_gather(c + 2, b)
        for b in range(min(2, n_chunks)):
            if wh[b] is not None:
                for h in wh[b]:
                    h.wait()

    return k


@functools.lru_cache(maxsize=None)
def _make_transpose(V: int, blk: int):
    grid = (V + blk - 1) // blk

    def body(x_ref, o_ref):
        o_ref[:, :D_MODEL] = x_ref[...].T
        o_ref[:, D_MODEL:] = jnp.zeros((blk, PAD_W - D_MODEL), jnp.float32)

    return pl.pallas_call(
        body,
        grid=(grid,),
        in_specs=[pl.BlockSpec((D_MODEL, blk), lambda i: (0, i))],
        out_specs=pl.BlockSpec((blk, PAD_W), lambda i: (i, 0)),
        out_shape=jax.ShapeDtypeStruct((V, PAD_W), jnp.float32),
    )


def kernel(tokens, W_E):
    BT, S = tokens.shape
    V = W_E.shape[0]
    flat = tokens.reshape(-1).astype(jnp.int32)
    padded = _make_transpose(V, 16384)(W_E.T)
    return _make_gather(BT, S, 400)(flat, padded)


# speedup vs baseline: 1.0130x; 1.0130x over previous
import functools

import jax
import jax.numpy as jnp
from jax import lax
from jax.experimental import pallas as pl
from jax.experimental.pallas import tpu as pltpu
from jax.experimental.pallas import tpu_sc as plsc

D_MODEL = 64
PAD_W = 128
NUM_CORES = 2
NUM_SUBCORES = 16
NW = NUM_CORES * NUM_SUBCORES


@functools.lru_cache(maxsize=None)
def _make_gather(BT: int, S: int, chunk: int):
    B = BT * S
    assert B % (NW * chunk) == 0 and chunk % S == 0
    b_per_w = B // NW
    n_chunks = b_per_w // chunk
    mesh = plsc.VectorSubcoreMesh(
        core_axis_name="c", subcore_axis_name="s",
        num_cores=NUM_CORES, num_subcores=NUM_SUBCORES)

    @functools.partial(
        pl.kernel,
        out_type=jax.ShapeDtypeStruct((BT, S, D_MODEL), jnp.float32),
        mesh=mesh,
        compiler_params=pltpu.CompilerParams(use_tc_tiling_on_sc=False),
        scratch_types=[
            pltpu.VMEM((b_per_w,), jnp.int32),
            pltpu.VMEM((chunk, PAD_W), jnp.float32),
            pltpu.VMEM((chunk, PAD_W), jnp.float32),
            pltpu.SemaphoreType.DMA,
            pltpu.SemaphoreType.DMA,
            pltpu.SemaphoreType.DMA,
            pltpu.SemaphoreType.DMA,
        ],
    )
    def k(idx_hbm, table_hbm, out_hbm,
          idx_v, rows0, rows1, g0, g1, w0, w1):
        wid = lax.axis_index("s") * NUM_CORES + lax.axis_index("c")
        base = wid * b_per_w
        rows = [rows0, rows1]
        gsem = [g0, g1]
        wsem = [w0, w1]

        pltpu.sync_copy(idx_hbm.at[pl.ds(base, b_per_w)], idx_v)

        def start_gather(c, b):
            return pltpu.async_copy(
                table_hbm.at[idx_v.at[pl.ds(c * chunk, chunk)]],
                rows[b], gsem[b])

        gh = [None, None]
        wh = [None, None]
        for c in range(min(2, n_chunks)):
            gh[c] = start_gather(c, c)
        for c in range(n_chunks):
            b = c & 1
            gh[b].wait()
            bbase = (base + c * chunk) // S
            wh[b] = [
                pltpu.async_copy(
                    rows[b].at[pl.ds(j * S, S), pl.ds(0, D_MODEL)],
                    out_hbm.at[bbase + j], wsem[b])
                for j in range(chunk // S)
            ]
            if c + 2 < n_chunks:
                for h in wh[b]:
                    h.wait()
                gh[b] = start_gather(c + 2, b)
        for b in range(min(2, n_chunks)):
            if wh[b] is not None:
                for h in wh[b]:
                    h.wait()

    return k


@functools.lru_cache(maxsize=None)
def _make_transpose(V: int, blk: int):
    grid = (V + blk - 1) // blk

    def body(x_ref, o_ref):
        o_ref[:, :D_MODEL] = x_ref[...].T
        o_ref[:, D_MODEL:] = jnp.zeros((blk, PAD_W - D_MODEL), jnp.float32)

    return pl.pallas_call(
        body,
        grid=(grid,),
        in_specs=[pl.BlockSpec((D_MODEL, blk), lambda i: (0, i))],
        out_specs=pl.BlockSpec((blk, PAD_W), lambda i: (i, 0)),
        out_shape=jax.ShapeDtypeStruct((V, PAD_W), jnp.float32),
    )


def kernel(tokens, W_E):
    BT, S = tokens.shape
    V = W_E.shape[0]
    flat = tokens.reshape(-1).astype(jnp.int32)
    padded = _make_transpose(V, 32768)(W_E.T)
    return _make_gather(BT, S, 400)(flat, padded)
